# baseline (speedup 1.0000x reference)
import functools

import jax
import jax.numpy as jnp
from jax import lax
from jax.experimental import pallas as pl
from jax.experimental.pallas import tpu as pltpu
from jax.experimental.pallas import tpu_sc as plsc

N = 10000
C = 128
E = 320000
NCORE = 2
NSUB = 16
NW = NCORE * NSUB
K = 128
CHUNKS = 80
EPT = CHUNKS * K
NA = N + 112
RPT = NA // NSUB
RB = 1000
GRID = N // RB

_PAIRS = [(1, 0), (2, 0), (2, 1), (3, 0), (3, 1), (3, 2)]


def _fill_zero(zref, rows, width):
    def body(i, _):
        for c8 in range(width // 16):
            zref[i, pl.ds(c8 * 16, 16)] = jnp.zeros((16,), jnp.float32)
        return 0
    lax.fori_loop(0, rows, body, 0)


def _zero_acc(acc, zero_v, base, width):
    for q in range(19):
        pltpu.sync_copy(zero_v, acc.at[pl.ds(base + q * 32, 32)])
    pltpu.sync_copy(zero_v.at[pl.ds(0, 24)], acc.at[pl.ds(base + 608, 24)])


def _make_scatter(nchunk):
    mesh = plsc.VectorSubcoreMesh(core_axis_name="c", subcore_axis_name="s")
    out_type = [jax.ShapeDtypeStruct((NCORE, NA, C), jnp.float32)
                for _ in range(nchunk)]
    HCH = CHUNKS // 2
    WIN = 48
    scratch = [
        pltpu.VMEM((WIN, K), jnp.int32),
        pltpu.VMEM((WIN, K), jnp.int32),
        pltpu.VMEM((K, C), jnp.float32),
        pltpu.VMEM((K, C), jnp.float32),
        pltpu.VMEM((32, C), jnp.float32),
        pltpu.VMEM_SHARED((NA, C), jnp.float32),
        pltpu.SemaphoreType.DMA,
        pltpu.SemaphoreType.DMA,
    ]

    @functools.partial(pl.kernel, out_type=out_type, mesh=mesh,
                       scratch_types=scratch)
    def scatter_kernel(*refs):
        ys = refs[:nchunk]
        rowp, colp = refs[nchunk], refs[nchunk + 1]
        outs = refs[nchunk + 2:nchunk + 2 + nchunk]
        (ridx, cidx, rows_a, rows_b, zero_v, acc,
         sem_a, sem_b) = refs[nchunk + 2 + nchunk:]

        cid = lax.axis_index("c")
        sid = lax.axis_index("s")
        wid = sid * NCORE + cid
        base = sid * RPT

        _fill_zero(zero_v, 32, C)

        for jc in range(nchunk):
            _zero_acc(acc, zero_v, base, C)
            plsc.subcore_barrier()

            y = ys[jc]
            for h in range(2):
                pltpu.sync_copy(rowp.at[wid, pl.ds(h * HCH, WIN)], ridx)
                pltpu.sync_copy(colp.at[wid, pl.ds(h * HCH, WIN)], cidx)
                pltpu.async_copy(y.at[ridx.at[0]], rows_a, sem_a)

                def pair(jj, _):
                    j = 2 * jj
                    pltpu.async_copy(y.at[ridx.at[j + 1]], rows_b, sem_b)
                    pltpu.make_async_copy(y.at[ridx.at[j]], rows_a,
                                          sem_a).wait()
                    pltpu.sync_copy(rows_a, acc.at[cidx.at[j]], add=True)
                    pltpu.async_copy(y.at[ridx.at[j + 2]], rows_a, sem_a)
                    pltpu.make_async_copy(y.at[ridx.at[j + 1]], rows_b,
                                          sem_b).wait()
                    pltpu.sync_copy(rows_b, acc.at[cidx.at[j + 1]], add=True)
                    return 0
                lax.fori_loop(0, HCH // 2, pair, 0)
                pltpu.make_async_copy(y.at[ridx.at[HCH]], rows_a,
                                      sem_a).wait()

            plsc.subcore_barrier()
            for off, sz in [(0, 128), (128, 128), (256, 128),
                            (384, 128), (512, 120)]:
                pltpu.sync_copy(acc.at[pl.ds(base + off, sz)],
                                rows_a.at[pl.ds(0, sz)])
                pltpu.sync_copy(rows_a.at[pl.ds(0, sz)],
                                outs[jc].at[cid, pl.ds(base + off, sz)])
            if jc + 1 < nchunk:
                plsc.subcore_barrier()

    return scatter_kernel


_scatter1 = _make_scatter(1)
_scatter3 = _make_scatter(3)


_rowblk = pl.BlockSpec((RB, C), lambda i: (i, 0))
_pblk = pl.BlockSpec((NCORE, RB, C), lambda i: (0, i, 0))


def _dinv_body(p_ref, out_ref):
    deg = p_ref[0, :, 0] + p_ref[1, :, 0]
    dinv = jnp.where(deg == 0, 0.0, 1.0 / jnp.where(deg == 0, 1.0, deg))
    out_ref[...] = jnp.broadcast_to(dinv[:, None], (RB, C))


def _dinv_call(partial):
    return pl.pallas_call(
        _dinv_body,
        grid=(GRID,),
        in_specs=[_pblk],
        out_specs=_rowblk,
        out_shape=jax.ShapeDtypeStruct((N, C), jnp.float32),
    )(partial)


def _scale_body(x_ref, d_ref, y_ref):
    y_ref[...] = x_ref[...] * d_ref[...]


def _scale_call(x, dinvb):
    return pl.pallas_call(
        _scale_body,
        grid=(GRID,),
        in_specs=[_rowblk, _rowblk],
        out_specs=_rowblk,
        out_shape=jax.ShapeDtypeStruct((N, C), jnp.float32),
    )(x, dinvb)


def _combine_body(cur_ref, p_ref, d_ref, nxt_ref, y_ref):
    s = 0.5 * (cur_ref[...] + p_ref[0] + p_ref[1])
    nxt_ref[...] = s
    y_ref[...] = d_ref[...] * s


def _combine_call(cur, partial, dinvb):
    return pl.pallas_call(
        _combine_body,
        grid=(GRID,),
        in_specs=[_rowblk, _pblk, _rowblk],
        out_specs=[_rowblk, _rowblk],
        out_shape=[jax.ShapeDtypeStruct((N, C), jnp.float32),
                   jax.ShapeDtypeStruct((N, C), jnp.float32)],
    )(cur, partial, dinvb)


def _make_wavelet(vs, nscale):
    nv = len(vs)

    def body(*refs):
        dls = refs[:17]
        w_ref, d_ref = refs[17], refs[18]
        souts = refs[19:19 + nv]
        youts = refs[19 + nv:]
        dq = [dls[t][...].astype(jnp.bfloat16).astype(jnp.float32)
              for t in range(17)]
        for k, v in enumerate(vs):
            acc = jnp.zeros((RB, C), jnp.float32)
            for t in range(17):
                wq = w_ref[v, t].astype(jnp.bfloat16).astype(jnp.float32)
                acc = acc + wq * dq[t]
            s = jnp.abs(acc)
            souts[k][...] = s
            if k < nscale:
                youts[k][...] = d_ref[...] * s

    def call(dl_list, wmat, dinvb):
        outs = pl.pallas_call(
            body,
            grid=(GRID,),
            in_specs=[_rowblk] * 17
            + [pl.BlockSpec(memory_space=pltpu.SMEM), _rowblk],
            out_specs=[_rowblk] * (nv + nscale),
            out_shape=[jax.ShapeDtypeStruct((N, C), jnp.float32)] * (nv + nscale),
        )(*dl_list, wmat, dinvb)
        return outs[:nv], outs[nv:]

    return call


_wavelet1 = _make_wavelet([0, 1, 2, 3], 3)
_wavelet2 = {
    0: _make_wavelet([1, 2, 3], 0),
    1: _make_wavelet([2, 3], 0),
    2: _make_wavelet([3], 0),
}


def _moments_body(f_ref, out_ref):
    f = f_ref[0]
    inv_n = 1.0 / N
    mu = jnp.sum(f, axis=0) * inv_n
    d = f - mu[None, :]
    d2 = d * d
    m2 = jnp.sum(d2, axis=0) * inv_n
    m3 = jnp.sum(d2 * d, axis=0) * inv_n
    m4 = jnp.sum(d2 * d2, axis=0) * inv_n
    den3 = m2 * jnp.sqrt(m2)
    skew = jnp.where(den3 > 0, m3 / jnp.where(den3 > 0, den3, 1.0), 0.0)
    skew = jnp.where(skew > 1e15, 0.0, skew)
    den4 = m2 * m2
    kurt = jnp.where(den4 > 0, m4 / jnp.where(den4 > 0, den4, 1.0) - 3.0, -3.0)
    kurt = jnp.where(kurt > 1e15, -3.0, kurt)
    out_ref[0] = jnp.stack([mu, m2, skew, kurt])


def _moments_call(blocks):
    return pl.pallas_call(
        _moments_body,
        grid=(11,),
        in_specs=[pl.BlockSpec((1, N, C), lambda j: (j, 0, 0))],
        out_specs=pl.BlockSpec((1, 4, C), lambda j: (j, 0, 0)),
        out_shape=jax.ShapeDtypeStruct((11, 4, C), jnp.float32),
    )(blocks)


def kernel(x, edge_index, wavelet_constructor):
    row, col = edge_index[0], edge_index[1]
    pad = NW * EPT - E
    rowp = jnp.concatenate(
        [row, jnp.zeros((pad,), jnp.int32)]).reshape(NW, CHUNKS, K)
    colp = jnp.concatenate(
        [col, jnp.full((pad,), N, jnp.int32)]).reshape(NW, CHUNKS, K)
    rowp = jnp.concatenate([rowp, jnp.zeros((NW, 8, K), jnp.int32)], axis=1)
    colp = jnp.concatenate([colp, jnp.full((NW, 8, K), N, jnp.int32)], axis=1)

    ones = jnp.ones((N, C), jnp.float32)
    deg_partial, = _scatter1(ones, rowp, colp)
    dinvb = _dinv_call(deg_partial)

    dl = [x]
    y = _scale_call(x, dinvb)
    for _ in range(16):
        p, = _scatter1(y, rowp, colp)
        nxt, y = _combine_call(dl[-1], p, dinvb)
        dl.append(nxt)

    s1, y2 = _wavelet1(dl, wavelet_constructor, dinvb)

    dl2 = [[s1[w]] for w in range(3)]
    ys = list(y2)
    for _ in range(16):
        ps = _scatter3(ys[0], ys[1], ys[2], rowp, colp)
        for w in range(3):
            nxt, yw = _combine_call(dl2[w][-1], ps[w], dinvb)
            dl2[w].append(nxt)
            ys[w] = yw

    s2 = []
    for w in range(3):
        outs, _ = _wavelet2[w](dl2[w], wavelet_constructor, dinvb)
        s2.extend(outs)
    s2 = [s2[0], s2[1], s2[3], s2[2], s2[4], s2[5]]

    blocks = jnp.stack([x, s1[0], s1[1], s1[2], s1[3]] + s2)
    mom = _moments_call(blocks)
    out = jnp.transpose(mom, (1, 0, 2)).reshape(1, 44 * C)
    return (out, wavelet_constructor)


# speedup vs baseline: 1.1360x; 1.1360x over previous
import functools

import jax
import jax.numpy as jnp
from jax import lax
from jax.experimental import pallas as pl
from jax.experimental.pallas import tpu as pltpu
from jax.experimental.pallas import tpu_sc as plsc

N = 10000
C = 128
E = 320000
NCORE = 2
NSUB = 16
NW = NCORE * NSUB
K = 128
CHUNKS = 80
EPT = CHUNKS * K
NA = N + 112
RPT = NA // NSUB
RB = 1000
GRID = N // RB

_PAIRS = [(1, 0), (2, 0), (2, 1), (3, 0), (3, 1), (3, 2)]


def _fill_zero(zref, rows, width):
    def body(i, _):
        for c8 in range(width // 16):
            zref[i, pl.ds(c8 * 16, 16)] = jnp.zeros((16,), jnp.float32)
        return 0
    lax.fori_loop(0, rows, body, 0)


def _zero_acc(acc, zero_v, base, width):
    for q in range(19):
        pltpu.sync_copy(zero_v, acc.at[pl.ds(base + q * 32, 32)])
    pltpu.sync_copy(zero_v.at[pl.ds(0, 24)], acc.at[pl.ds(base + 608, 24)])


def _make_scatter(nchunk):
    mesh = plsc.VectorSubcoreMesh(core_axis_name="c", subcore_axis_name="s")
    out_type = [jax.ShapeDtypeStruct((NCORE, NA, C), jnp.float32)
                for _ in range(nchunk)]
    scratch = [
        pltpu.VMEM((CHUNKS, K), jnp.int32),
        pltpu.VMEM((CHUNKS, K), jnp.int32),
        pltpu.VMEM((K, C), jnp.float32),
        pltpu.VMEM((32, C), jnp.float32),
        pltpu.VMEM_SHARED((NA, C), jnp.float32),
        pltpu.SemaphoreType.DMA,
    ]

    @functools.partial(pl.kernel, out_type=out_type, mesh=mesh,
                       scratch_types=scratch)
    def scatter_kernel(*refs):
        ys = refs[:nchunk]
        rowp, colp = refs[nchunk], refs[nchunk + 1]
        outs = refs[nchunk + 2:nchunk + 2 + nchunk]
        ridx, cidx, rows_v, zero_v, acc, sem = refs[nchunk + 2 + nchunk:]

        cid = lax.axis_index("c")
        sid = lax.axis_index("s")
        wid = sid * NCORE + cid
        base = sid * RPT

        pltpu.sync_copy(rowp.at[wid], ridx)
        pltpu.sync_copy(colp.at[wid], cidx)
        _fill_zero(zero_v, 32, C)

        for jc in range(nchunk):
            _zero_acc(acc, zero_v, base, C)
            plsc.subcore_barrier()

            y = ys[jc]

            def body(j, _):
                pltpu.async_copy(y.at[ridx.at[j]], rows_v, sem).wait()
                pltpu.sync_copy(rows_v, acc.at[cidx.at[j]], add=True)
                return 0
            lax.fori_loop(0, CHUNKS, body, 0)

            plsc.subcore_barrier()
            for off, sz in [(0, 128), (128, 128), (256, 128),
                            (384, 128), (512, 120)]:
                pltpu.sync_copy(acc.at[pl.ds(base + off, sz)],
                                rows_v.at[pl.ds(0, sz)])
                pltpu.sync_copy(rows_v.at[pl.ds(0, sz)],
                                outs[jc].at[cid, pl.ds(base + off, sz)])
            if jc + 1 < nchunk:
                plsc.subcore_barrier()

    return scatter_kernel


_scatter1 = _make_scatter(1)
_scatter3 = _make_scatter(3)


_rowblk = pl.BlockSpec((RB, C), lambda i: (i, 0))
_pblk = pl.BlockSpec((NCORE, RB, C), lambda i: (0, i, 0))


def _dinv_body(p_ref, out_ref):
    deg = p_ref[0, :, 0] + p_ref[1, :, 0]
    dinv = jnp.where(deg == 0, 0.0, 1.0 / jnp.where(deg == 0, 1.0, deg))
    out_ref[...] = jnp.broadcast_to(dinv[:, None], (RB, C))


def _dinv_call(partial):
    return pl.pallas_call(
        _dinv_body,
        grid=(GRID,),
        in_specs=[_pblk],
        out_specs=_rowblk,
        out_shape=jax.ShapeDtypeStruct((N, C), jnp.float32),
    )(partial)


def _scale_body(x_ref, d_ref, y_ref):
    y_ref[...] = x_ref[...] * d_ref[...]


def _scale_call(x, dinvb):
    return pl.pallas_call(
        _scale_body,
        grid=(GRID,),
        in_specs=[_rowblk, _rowblk],
        out_specs=_rowblk,
        out_shape=jax.ShapeDtypeStruct((N, C), jnp.float32),
    )(x, dinvb)


def _combine_body(cur_ref, p_ref, d_ref, nxt_ref, y_ref):
    s = 0.5 * (cur_ref[...] + p_ref[0] + p_ref[1])
    nxt_ref[...] = s
    y_ref[...] = d_ref[...] * s


def _combine_call(cur, partial, dinvb):
    return pl.pallas_call(
        _combine_body,
        grid=(GRID,),
        in_specs=[_rowblk, _pblk, _rowblk],
        out_specs=[_rowblk, _rowblk],
        out_shape=[jax.ShapeDtypeStruct((N, C), jnp.float32),
                   jax.ShapeDtypeStruct((N, C), jnp.float32)],
    )(cur, partial, dinvb)


def _make_wavelet(vs, nscale):
    nv = len(vs)

    def body(*refs):
        dls = refs[:17]
        w_ref, d_ref = refs[17], refs[18]
        souts = refs[19:19 + nv]
        youts = refs[19 + nv:]
        dq = [dls[t][...].astype(jnp.bfloat16).astype(jnp.float32)
              for t in range(17)]
        for k, v in enumerate(vs):
            acc = jnp.zeros((RB, C), jnp.float32)
            for t in range(17):
                wq = w_ref[v, t].astype(jnp.bfloat16).astype(jnp.float32)
                acc = acc + wq * dq[t]
            s = jnp.abs(acc)
            souts[k][...] = s
            if k < nscale:
                youts[k][...] = d_ref[...] * s

    def call(dl_list, wmat, dinvb):
        outs = pl.pallas_call(
            body,
            grid=(GRID,),
            in_specs=[_rowblk] * 17
            + [pl.BlockSpec(memory_space=pltpu.SMEM), _rowblk],
            out_specs=[_rowblk] * (nv + nscale),
            out_shape=[jax.ShapeDtypeStruct((N, C), jnp.float32)] * (nv + nscale),
        )(*dl_list, wmat, dinvb)
        return outs[:nv], outs[nv:]

    return call


_wavelet1 = _make_wavelet([0, 1, 2, 3], 3)
_wavelet2 = {
    0: _make_wavelet([1, 2, 3], 0),
    1: _make_wavelet([2, 3], 0),
    2: _make_wavelet([3], 0),
}


def _moments_body(f_ref, out_ref):
    f = f_ref[0]
    inv_n = 1.0 / N
    mu = jnp.sum(f, axis=0) * inv_n
    d = f - mu[None, :]
    d2 = d * d
    m2 = jnp.sum(d2, axis=0) * inv_n
    m3 = jnp.sum(d2 * d, axis=0) * inv_n
    m4 = jnp.sum(d2 * d2, axis=0) * inv_n
    den3 = m2 * jnp.sqrt(m2)
    skew = jnp.where(den3 > 0, m3 / jnp.where(den3 > 0, den3, 1.0), 0.0)
    skew = jnp.where(skew > 1e15, 0.0, skew)
    den4 = m2 * m2
    kurt = jnp.where(den4 > 0, m4 / jnp.where(den4 > 0, den4, 1.0) - 3.0, -3.0)
    kurt = jnp.where(kurt > 1e15, -3.0, kurt)
    out_ref[0] = jnp.stack([mu, m2, skew, kurt])


def _moments_call(blocks):
    return pl.pallas_call(
        _moments_body,
        grid=(11,),
        in_specs=[pl.BlockSpec((1, N, C), lambda j: (j, 0, 0))],
        out_specs=pl.BlockSpec((1, 4, C), lambda j: (j, 0, 0)),
        out_shape=jax.ShapeDtypeStruct((11, 4, C), jnp.float32),
    )(blocks)


def kernel(x, edge_index, wavelet_constructor):
    row, col = edge_index[0], edge_index[1]
    pad = NW * EPT - E
    rowp = jnp.concatenate(
        [row, jnp.zeros((pad,), jnp.int32)]).reshape(NW, CHUNKS, K)
    colp = jnp.concatenate(
        [col, jnp.full((pad,), N, jnp.int32)]).reshape(NW, CHUNKS, K)

    ones = jnp.ones((N, C), jnp.float32)
    deg_partial, = _scatter1(ones, rowp, colp)
    dinvb = _dinv_call(deg_partial)

    dl = [x]
    y = _scale_call(x, dinvb)
    for _ in range(16):
        p, = _scatter1(y, rowp, colp)
        nxt, y = _combine_call(dl[-1], p, dinvb)
        dl.append(nxt)

    s1, y2 = _wavelet1(dl, wavelet_constructor, dinvb)

    dl2 = [[s1[w]] for w in range(3)]
    ys = list(y2)
    for _ in range(16):
        ps = _scatter3(ys[0], ys[1], ys[2], rowp, colp)
        for w in range(3):
            nxt, yw = _combine_call(dl2[w][-1], ps[w], dinvb)
            dl2[w].append(nxt)
            ys[w] = yw

    s2 = []
    for w in range(3):
        outs, _ = _wavelet2[w](dl2[w], wavelet_constructor, dinvb)
        s2.extend(outs)
    s2 = [s2[0], s2[1], s2[3], s2[2], s2[4], s2[5]]

    blocks = jnp.stack([x, s1[0], s1[1], s1[2], s1[3]] + s2)
    mom = _moments_call(blocks)
    out = jnp.transpose(mom, (1, 0, 2)).reshape(1, 44 * C)
    return (out, wavelet_constructor)


# speedup vs baseline: 1.7706x; 1.5587x over previous
import functools

import jax
import jax.numpy as jnp
from jax import lax
from jax.experimental import pallas as pl
from jax.experimental.pallas import tpu as pltpu
from jax.experimental.pallas import tpu_sc as plsc

N = 10000
C = 128
E = 320000
NCORE = 2
NSUB = 16
NW = NCORE * NSUB
K = 128
CHUNKS = 79
EPT = CHUNKS * K
NA = N + 112
RPT = NA // NSUB
RB = 1000
GRID = N // RB

_PAIRS = [(1, 0), (2, 0), (2, 1), (3, 0), (3, 1), (3, 2)]


def _fill_zero(zref, rows, width):
    def body(i, _):
        for c8 in range(width // 16):
            zref[i, pl.ds(c8 * 16, 16)] = jnp.zeros((16,), jnp.float32)
        return 0
    lax.fori_loop(0, rows, body, 0)


def _zero_acc(acc, zero_v, base, width):
    for q in range(9):
        pltpu.sync_copy(zero_v, acc.at[pl.ds(base + q * 64, 64)])
    pltpu.sync_copy(zero_v.at[pl.ds(0, 56)], acc.at[pl.ds(base + 576, 56)])


def _make_scatter(nchunk):
    mesh = plsc.VectorSubcoreMesh(core_axis_name="c", subcore_axis_name="s")
    out_type = [jax.ShapeDtypeStruct((NCORE, NA, C), jnp.float32)
                for _ in range(nchunk)]
    scratch = [
        pltpu.VMEM((CHUNKS, K), jnp.int32),
        pltpu.VMEM((CHUNKS, K), jnp.int32),
        pltpu.VMEM((K, C), jnp.float32),
        pltpu.VMEM((64, C), jnp.float32),
        pltpu.VMEM_SHARED((NA, C), jnp.float32),
        pltpu.SemaphoreType.DMA,
    ]

    @functools.partial(pl.kernel, out_type=out_type, mesh=mesh,
                       scratch_types=scratch)
    def scatter_kernel(*refs):
        ys = refs[:nchunk]
        rowp, colp = refs[nchunk], refs[nchunk + 1]
        outs = refs[nchunk + 2:nchunk + 2 + nchunk]
        ridx, cidx, rows_v, zero_v, acc, sem = refs[nchunk + 2 + nchunk:]

        cid = lax.axis_index("c")
        sid = lax.axis_index("s")
        wid = sid * NCORE + cid
        base = sid * RPT

        pltpu.sync_copy(rowp.at[wid], ridx)
        pltpu.sync_copy(colp.at[wid], cidx)
        _fill_zero(zero_v, 64, C)

        for jc in range(nchunk):
            _zero_acc(acc, zero_v, base, C)
            plsc.subcore_barrier()

            y = ys[jc]

            def body(j, _):
                pltpu.async_copy(y.at[ridx.at[j]], rows_v, sem).wait()
                pltpu.sync_copy(rows_v, acc.at[cidx.at[j]], add=True)
                return 0
            lax.fori_loop(0, CHUNKS, body, 0)

            plsc.subcore_barrier()
            for off, sz in [(0, 128), (128, 128), (256, 128),
                            (384, 128), (512, 120)]:
                pltpu.sync_copy(acc.at[pl.ds(base + off, sz)],
                                rows_v.at[pl.ds(0, sz)])
                pltpu.sync_copy(rows_v.at[pl.ds(0, sz)],
                                outs[jc].at[cid, pl.ds(base + off, sz)])
            if jc + 1 < nchunk:
                plsc.subcore_barrier()

    return scatter_kernel


_scatter1 = _make_scatter(1)
_scatter3 = _make_scatter(3)


_rowblk = pl.BlockSpec((RB, C), lambda i: (i, 0))
_pblk = pl.BlockSpec((NCORE, RB, C), lambda i: (0, i, 0))


def _dinv_body(p_ref, out_ref):
    deg = p_ref[0, :, 0] + p_ref[1, :, 0]
    dinv = jnp.where(deg == 0, 0.0, 1.0 / jnp.where(deg == 0, 1.0, deg))
    out_ref[...] = jnp.broadcast_to(dinv[:, None], (RB, C))


def _dinv_call(partial):
    return pl.pallas_call(
        _dinv_body,
        grid=(GRID,),
        in_specs=[_pblk],
        out_specs=_rowblk,
        out_shape=jax.ShapeDtypeStruct((N, C), jnp.float32),
    )(partial)


def _scale_body(x_ref, d_ref, y_ref):
    y_ref[...] = x_ref[...] * d_ref[...]


def _scale_call(x, dinvb):
    return pl.pallas_call(
        _scale_body,
        grid=(GRID,),
        in_specs=[_rowblk, _rowblk],
        out_specs=_rowblk,
        out_shape=jax.ShapeDtypeStruct((N, C), jnp.float32),
    )(x, dinvb)


def _combine_body(cur_ref, p_ref, d_ref, nxt_ref, y_ref):
    s = 0.5 * (cur_ref[...] + p_ref[0] + p_ref[1])
    nxt_ref[...] = s
    y_ref[...] = d_ref[...] * s


def _combine_call(cur, partial, dinvb):
    return pl.pallas_call(
        _combine_body,
        grid=(GRID,),
        in_specs=[_rowblk, _pblk, _rowblk],
        out_specs=[_rowblk, _rowblk],
        out_shape=[jax.ShapeDtypeStruct((N, C), jnp.float32),
                   jax.ShapeDtypeStruct((N, C), jnp.float32)],
    )(cur, partial, dinvb)


def _make_wavelet(vs, nscale):
    nv = len(vs)

    def body(*refs):
        dls = refs[:17]
        w_ref, d_ref = refs[17], refs[18]
        souts = refs[19:19 + nv]
        youts = refs[19 + nv:]
        dq = [dls[t][...].astype(jnp.bfloat16).astype(jnp.float32)
              for t in range(17)]
        for k, v in enumerate(vs):
            acc = jnp.zeros((RB, C), jnp.float32)
            for t in range(17):
                wq = w_ref[v, t].astype(jnp.bfloat16).astype(jnp.float32)
                acc = acc + wq * dq[t]
            s = jnp.abs(acc)
            souts[k][...] = s
            if k < nscale:
                youts[k][...] = d_ref[...] * s

    def call(dl_list, wmat, dinvb):
        outs = pl.pallas_call(
            body,
            grid=(GRID,),
            in_specs=[_rowblk] * 17
            + [pl.BlockSpec(memory_space=pltpu.SMEM), _rowblk],
            out_specs=[_rowblk] * (nv + nscale),
            out_shape=[jax.ShapeDtypeStruct((N, C), jnp.float32)] * (nv + nscale),
        )(*dl_list, wmat, dinvb)
        return outs[:nv], outs[nv:]

    return call


_wavelet1 = _make_wavelet([0, 1, 2, 3], 3)
_wavelet2 = {
    0: _make_wavelet([1, 2, 3], 0),
    1: _make_wavelet([2, 3], 0),
    2: _make_wavelet([3], 0),
}


def _moments_body(f_ref, out_ref):
    f = f_ref[0]
    inv_n = 1.0 / N
    mu = jnp.sum(f, axis=0) * inv_n
    d = f - mu[None, :]
    d2 = d * d
    m2 = jnp.sum(d2, axis=0) * inv_n
    m3 = jnp.sum(d2 * d, axis=0) * inv_n
    m4 = jnp.sum(d2 * d2, axis=0) * inv_n
    den3 = m2 * jnp.sqrt(m2)
    skew = jnp.where(den3 > 0, m3 / jnp.where(den3 > 0, den3, 1.0), 0.0)
    skew = jnp.where(skew > 1e15, 0.0, skew)
    den4 = m2 * m2
    kurt = jnp.where(den4 > 0, m4 / jnp.where(den4 > 0, den4, 1.0) - 3.0, -3.0)
    kurt = jnp.where(kurt > 1e15, -3.0, kurt)
    out_ref[0] = jnp.stack([mu, m2, skew, kurt])


def _moments_call(blocks):
    return pl.pallas_call(
        _moments_body,
        grid=(11,),
        in_specs=[pl.BlockSpec((1, N, C), lambda j: (j, 0, 0))],
        out_specs=pl.BlockSpec((1, 4, C), lambda j: (j, 0, 0)),
        out_shape=jax.ShapeDtypeStruct((11, 4, C), jnp.float32),
    )(blocks)


def kernel(x, edge_index, wavelet_constructor):
    row, col = edge_index[0], edge_index[1]
    pad = NW * EPT - E
    rowp = jnp.concatenate(
        [row, jnp.zeros((pad,), jnp.int32)]).reshape(NW, CHUNKS, K)
    colp = jnp.concatenate(
        [col, jnp.full((pad,), N, jnp.int32)]).reshape(NW, CHUNKS, K)

    ones = jnp.ones((N, C), jnp.float32)
    deg_partial, = _scatter1(ones, rowp, colp)
    dinvb = _dinv_call(deg_partial)

    dl = [x]
    y = _scale_call(x, dinvb)
    for _ in range(16):
        p, = _scatter1(y, rowp, colp)
        nxt, y = _combine_call(dl[-1], p, dinvb)
        dl.append(nxt)

    s1, y2 = _wavelet1(dl, wavelet_constructor, dinvb)

    dl2 = [[s1[w]] for w in range(3)]
    ys = list(y2)
    for _ in range(16):
        ps = _scatter3(ys[0], ys[1], ys[2], rowp, colp)
        for w in range(3):
            nxt, yw = _combine_call(dl2[w][-1], ps[w], dinvb)
            dl2[w].append(nxt)
            ys[w] = yw

    s2 = []
    for w in range(3):
        outs, _ = _wavelet2[w](dl2[w], wavelet_constructor, dinvb)
        s2.extend(outs)
    s2 = [s2[0], s2[1], s2[3], s2[2], s2[4], s2[5]]

    blocks = jnp.stack([x, s1[0], s1[1], s1[2], s1[3]] + s2)
    mom = _moments_call(blocks)
    out = jnp.transpose(mom, (1, 0, 2)).reshape(1, 44 * C)
    return (out, wavelet_constructor)
